# baseline (speedup 1.0000x reference)
import jax
import jax.numpy as jnp
from jax.experimental import pallas as pl
from jax.experimental.pallas import tpu as pltpu

NUM_USERS = 4096
NUM_ITEMS = 12288
N = NUM_USERS + NUM_ITEMS
EMB = 64
HID = 32
BATCH = 4096

BR = 256
NBU = NUM_USERS // BR
NB = N // BR
NBI = NB - NBU


def _gcn_body(adj_ref, ue_ref, ie_ref, uid_ref, iid_ref, w1_ref, b1_ref,
              w2_ref, b2_ref, wp1_ref, bp1_ref, wp2_ref, bp2_ref,
              u_ref, it_ref, s_ref, y1_s, y2_s, gu_s, gi_s):
    l = pl.program_id(0)
    i = pl.program_id(1)

    @pl.when((l == 0) & (i == 0))
    def _prep():
        y1_s[0:NUM_USERS, :] = (
            jnp.dot(ue_ref[...], w1_ref[...], preferred_element_type=jnp.float32)
            + b1_ref[...]
        )
        y1_s[NUM_USERS:, :] = (
            jnp.dot(ie_ref[...], w1_ref[...], preferred_element_type=jnp.float32)
            + b1_ref[...]
        )

    @pl.when(l == 0)
    def _layer1():
        acc = jnp.dot(adj_ref[...], y1_s[...], preferred_element_type=jnp.float32)
        h = jnp.maximum(acc, 0.0)
        y2_s[pl.ds(i * BR, BR), :] = (
            jnp.dot(h, w2_ref[...], preferred_element_type=jnp.float32)
            + b2_ref[...]
        )

    @pl.when(l == 1)
    def _layer2():
        acc = jnp.dot(adj_ref[...], y2_s[...], preferred_element_type=jnp.float32)
        h = jnp.maximum(acc, 0.0)

        row0 = jax.lax.broadcasted_iota(jnp.int16, (BR, BATCH), 0)

        @pl.when(i < NBI)
        def _():
            it_ref[...] = h
            local = (iid_ref[...] - i * BR).astype(jnp.int16)
            sel = (row0 == local).astype(jnp.bfloat16)
            contrib = jax.lax.dot_general(
                sel, h, (((0,), (0,)), ((), ())),
                preferred_element_type=jnp.float32)

            @pl.when(i == 0)
            def _():
                gi_s[...] = contrib

            @pl.when(i > 0)
            def _():
                gi_s[...] += contrib

        @pl.when(i >= NBI)
        def _():
            u_ref[...] = h
            local = (uid_ref[...] - (i - NBI) * BR).astype(jnp.int16)
            sel = (row0 == local).astype(jnp.bfloat16)
            contrib = jax.lax.dot_general(
                sel, h, (((0,), (0,)), ((), ())),
                preferred_element_type=jnp.float32)

            @pl.when(i == NBI)
            def _():
                gu_s[...] = contrib

            @pl.when(i > NBI)
            def _():
                gu_s[...] += contrib

    @pl.when((l == 1) & (i == NBI))
    def _head_item_part():
        hi = (
            jnp.dot(gi_s[...], wp1_ref[EMB:, :],
                    preferred_element_type=jnp.float32)
            + bp1_ref[...]
        )
        gi_s[:, 0:HID] = hi

    @pl.when((l == 1) & (i == NB - 1))
    def _head():
        hid = jnp.maximum(
            jnp.dot(gu_s[...], wp1_ref[0:EMB, :],
                    preferred_element_type=jnp.float32)
            + gi_s[:, 0:HID],
            0.0,
        )
        z = jnp.sum(hid * wp2_ref[...], axis=1) + bp2_ref[0, 0]
        s_ref[...] = jax.nn.sigmoid(z)


def kernel(user_ids, item_ids, adj_matrix, user_emb, item_emb,
           W1, b1, W2, b2, Wp1, bp1, Wp2, bp2):
    f32 = jnp.float32
    grid = (2, NB)

    final_user_emb, final_item_emb, scores = pl.pallas_call(
        _gcn_body,
        grid=grid,
        in_specs=[
            pl.BlockSpec(
                (BR, N),
                lambda l, i: (
                    jnp.where(l == 0, i,
                              jnp.where(i < NBI, i + NBU, i - NBI)),
                    0,
                ),
            ),
            pl.BlockSpec((NUM_USERS, EMB), lambda l, i: (0, 0)),
            pl.BlockSpec((NUM_ITEMS, EMB), lambda l, i: (0, 0)),
            pl.BlockSpec((1, BATCH), lambda l, i: (0, 0)),
            pl.BlockSpec((1, BATCH), lambda l, i: (0, 0)),
            pl.BlockSpec((EMB, HID), lambda l, i: (0, 0)),
            pl.BlockSpec((1, HID), lambda l, i: (0, 0)),
            pl.BlockSpec((HID, EMB), lambda l, i: (0, 0)),
            pl.BlockSpec((1, EMB), lambda l, i: (0, 0)),
            pl.BlockSpec((2 * EMB, HID), lambda l, i: (0, 0)),
            pl.BlockSpec((1, HID), lambda l, i: (0, 0)),
            pl.BlockSpec((1, HID), lambda l, i: (0, 0)),
            pl.BlockSpec((1, 1), lambda l, i: (0, 0)),
        ],
        out_specs=[
            pl.BlockSpec((BR, EMB), lambda l, i: (l * jnp.maximum(i - NBI, 0), 0)),
            pl.BlockSpec((BR, EMB), lambda l, i: (l * jnp.minimum(i, NBI - 1), 0)),
            pl.BlockSpec((BATCH,), lambda l, i: (0,)),
        ],
        out_shape=[
            jax.ShapeDtypeStruct((NUM_USERS, EMB), f32),
            jax.ShapeDtypeStruct((NUM_ITEMS, EMB), f32),
            jax.ShapeDtypeStruct((BATCH,), f32),
        ],
        scratch_shapes=[
            pltpu.VMEM((N, HID), f32),
            pltpu.VMEM((N, EMB), f32),
            pltpu.VMEM((BATCH, EMB), f32),
            pltpu.VMEM((BATCH, EMB), f32),
        ],
        compiler_params=pltpu.CompilerParams(
            dimension_semantics=("arbitrary", "arbitrary"),
            vmem_limit_bytes=64 * 1024 * 1024,
        ),
    )(adj_matrix, user_emb, item_emb,
      user_ids.astype(jnp.int32).reshape(1, BATCH),
      item_ids.astype(jnp.int32).reshape(1, BATCH),
      W1, b1.reshape(1, HID), W2, b2.reshape(1, EMB),
      Wp1, bp1.reshape(1, HID), Wp2.reshape(1, HID), bp2.reshape(1, 1))

    return (scores, final_user_emb, final_item_emb)


# speedup vs baseline: 1.0012x; 1.0012x over previous
import jax
import jax.numpy as jnp
from jax.experimental import pallas as pl
from jax.experimental.pallas import tpu as pltpu

NUM_USERS = 4096
NUM_ITEMS = 12288
N = NUM_USERS + NUM_ITEMS
EMB = 64
HID = 32
BATCH = 4096

BR = 256
NBU = NUM_USERS // BR
NB = N // BR
NBI = NB - NBU


def _gcn_body(adj_ref, ue_ref, ie_ref, uid_ref, iid_ref, w1_ref, b1_ref,
              w2_ref, b2_ref, wp1_ref, bp1_ref, wp2_ref, bp2_ref,
              u_ref, it_ref, s_ref, y1_s, y2_s, gu_s, gi_s):
    l = pl.program_id(0)
    i = pl.program_id(1)

    @pl.when((l == 0) & (i == 0))
    def _prep():
        y1_s[0:NUM_USERS, :] = (
            jnp.dot(ue_ref[...], w1_ref[...], preferred_element_type=jnp.float32)
            + b1_ref[...]
        )
        y1_s[NUM_USERS:, :] = (
            jnp.dot(ie_ref[...], w1_ref[...], preferred_element_type=jnp.float32)
            + b1_ref[...]
        )

    @pl.when(l == 0)
    def _layer1():
        acc = jnp.dot(adj_ref[...], y1_s[...], preferred_element_type=jnp.float32)
        h = jnp.maximum(acc, 0.0)
        y2_s[pl.ds(i * BR, BR), :] = (
            jnp.dot(h, w2_ref[...], preferred_element_type=jnp.float32)
            + b2_ref[...]
        )

    @pl.when(l == 1)
    def _layer2():
        acc = jnp.dot(adj_ref[...], y2_s[...], preferred_element_type=jnp.float32)
        h = jnp.maximum(acc, 0.0)

        row0 = jax.lax.broadcasted_iota(jnp.int16, (BR, BATCH), 0)

        @pl.when(i < NBU)
        def _():
            u_ref[...] = h
            local = (uid_ref[...] - i * BR).astype(jnp.int16)
            sel = (row0 == local).astype(jnp.bfloat16)
            contrib = jax.lax.dot_general(
                sel, h, (((0,), (0,)), ((), ())),
                preferred_element_type=jnp.float32)

            @pl.when(i == 0)
            def _():
                gu_s[...] = contrib

            @pl.when(i > 0)
            def _():
                gu_s[...] += contrib

        @pl.when(i >= NBU)
        def _():
            it_ref[...] = h
            local = (iid_ref[...] - (i - NBU) * BR).astype(jnp.int16)
            sel = (row0 == local).astype(jnp.bfloat16)
            contrib = jax.lax.dot_general(
                sel, h, (((0,), (0,)), ((), ())),
                preferred_element_type=jnp.float32)

            @pl.when(i == NBU)
            def _():
                gi_s[...] = contrib

            @pl.when(i > NBU)
            def _():
                gi_s[...] += contrib

    @pl.when((l == 1) & (i == NB - 1))
    def _head():
        hid = jnp.maximum(
            jnp.dot(gu_s[...], wp1_ref[0:EMB, :],
                    preferred_element_type=jnp.float32)
            + jnp.dot(gi_s[...], wp1_ref[EMB:, :],
                      preferred_element_type=jnp.float32)
            + bp1_ref[...],
            0.0,
        )
        z = jnp.sum(hid * wp2_ref[...], axis=1) + bp2_ref[0, 0]
        s_ref[...] = jax.nn.sigmoid(z)


def kernel(user_ids, item_ids, adj_matrix, user_emb, item_emb,
           W1, b1, W2, b2, Wp1, bp1, Wp2, bp2):
    f32 = jnp.float32
    grid = (2, NB)

    final_user_emb, final_item_emb, scores = pl.pallas_call(
        _gcn_body,
        grid=grid,
        in_specs=[
            pl.BlockSpec((BR, N), lambda l, i: (i, 0)),
            pl.BlockSpec((NUM_USERS, EMB), lambda l, i: (0, 0)),
            pl.BlockSpec((NUM_ITEMS, EMB), lambda l, i: (0, 0)),
            pl.BlockSpec((1, BATCH), lambda l, i: (0, 0)),
            pl.BlockSpec((1, BATCH), lambda l, i: (0, 0)),
            pl.BlockSpec((EMB, HID), lambda l, i: (0, 0)),
            pl.BlockSpec((1, HID), lambda l, i: (0, 0)),
            pl.BlockSpec((HID, EMB), lambda l, i: (0, 0)),
            pl.BlockSpec((1, EMB), lambda l, i: (0, 0)),
            pl.BlockSpec((2 * EMB, HID), lambda l, i: (0, 0)),
            pl.BlockSpec((1, HID), lambda l, i: (0, 0)),
            pl.BlockSpec((1, HID), lambda l, i: (0, 0)),
            pl.BlockSpec((1, 1), lambda l, i: (0, 0)),
        ],
        out_specs=[
            pl.BlockSpec((BR, EMB), lambda l, i: (l * jnp.minimum(i, NBU - 1), 0)),
            pl.BlockSpec((BR, EMB), lambda l, i: (l * jnp.maximum(i - NBU, 0), 0)),
            pl.BlockSpec((BATCH,), lambda l, i: (0,)),
        ],
        out_shape=[
            jax.ShapeDtypeStruct((NUM_USERS, EMB), f32),
            jax.ShapeDtypeStruct((NUM_ITEMS, EMB), f32),
            jax.ShapeDtypeStruct((BATCH,), f32),
        ],
        scratch_shapes=[
            pltpu.VMEM((N, HID), f32),
            pltpu.VMEM((N, EMB), f32),
            pltpu.VMEM((BATCH, EMB), f32),
            pltpu.VMEM((BATCH, EMB), f32),
        ],
        compiler_params=pltpu.CompilerParams(
            dimension_semantics=("arbitrary", "arbitrary"),
            vmem_limit_bytes=64 * 1024 * 1024,
        ),
    )(adj_matrix, user_emb, item_emb,
      user_ids.astype(jnp.int32).reshape(1, BATCH),
      item_ids.astype(jnp.int32).reshape(1, BATCH),
      W1, b1.reshape(1, HID), W2, b2.reshape(1, EMB),
      Wp1, bp1.reshape(1, HID), Wp2.reshape(1, HID), bp2.reshape(1, 1))

    return (scores, final_user_emb, final_item_emb)
